# baseline (speedup 1.0000x reference)
import functools

import jax
import jax.numpy as jnp
from jax import lax
from jax.experimental import pallas as pl
from jax.experimental.pallas import tpu as pltpu
from jax.experimental.pallas import tpu_sc as plsc

N = 10000
E = 320000
D = 128
NC = 2
NS = 16
NW = NC * NS
CHUNK = 80
CHUNKS_PER_W = E // (NW * CHUNK)
ROW_BLK = 624
ROW_BLK_LAST = N - (NS - 1) * ROW_BLK
SCH = 128
EPW = E // NW
EPT = SCH * (-(-EPW // SCH))
SCPW = EPT // SCH
NPAD = N + 8
SROW_BLK_LAST = NPAD - (NS - 1) * ROW_BLK
DEG_W = 16

_MESH = plsc.VectorSubcoreMesh(
    core_axis_name="c", subcore_axis_name="s", num_cores=NC, num_subcores=NS
)


@functools.partial(
    pl.kernel,
    out_type=jax.ShapeDtypeStruct((NC, N, DEG_W), jnp.float32),
    mesh=_MESH,
    compiler_params=pltpu.CompilerParams(use_tc_tiling_on_sc=False),
    scratch_types=[
        pltpu.VMEM((CHUNKS_PER_W, CHUNK), jnp.int32),
        pltpu.VMEM((CHUNK, DEG_W), jnp.float32),
        pltpu.VMEM_SHARED((N, DEG_W), jnp.float32),
        pltpu.SemaphoreType.DMA,
    ],
)
def _deg_kernel(dst_hbm, ones_hbm, zeros_hbm, out_hbm, idx_v, ones_v, acc_sh,
                sem):
    c = lax.axis_index("c")
    s = lax.axis_index("s")
    wid = s * NC + c
    r0 = s * ROW_BLK

    @pl.when(s < NS - 1)
    def _():
        pltpu.sync_copy(zeros_hbm.at[pl.ds(0, ROW_BLK)],
                        acc_sh.at[pl.ds(r0, ROW_BLK)])

    @pl.when(s == NS - 1)
    def _():
        pltpu.sync_copy(zeros_hbm, acc_sh.at[pl.ds(r0, ROW_BLK_LAST)])

    pltpu.sync_copy(ones_hbm, ones_v)
    pltpu.sync_copy(dst_hbm.at[wid], idx_v)
    plsc.subcore_barrier()

    WINDOW = 8

    def body(i, carry):
        @pl.when(i >= WINDOW)
        def _():
            pltpu.make_async_copy(ones_v, acc_sh.at[idx_v.at[0]], sem).wait()

        pltpu.async_copy(ones_v, acc_sh.at[idx_v.at[i]], sem, add=True)
        return carry

    lax.fori_loop(0, CHUNKS_PER_W, body, 0)

    def drain(i, carry):
        pltpu.make_async_copy(ones_v, acc_sh.at[idx_v.at[0]], sem).wait()
        return carry

    lax.fori_loop(0, WINDOW, drain, 0)
    plsc.subcore_barrier()

    @pl.when(s < NS - 1)
    def _():
        pltpu.sync_copy(acc_sh.at[pl.ds(r0, ROW_BLK)],
                        out_hbm.at[c, pl.ds(r0, ROW_BLK)])

    @pl.when(s == NS - 1)
    def _():
        pltpu.sync_copy(acc_sh.at[pl.ds(r0, ROW_BLK_LAST)],
                        out_hbm.at[c, pl.ds(r0, ROW_BLK_LAST)])


@functools.partial(
    pl.kernel,
    out_type=jax.ShapeDtypeStruct((NC, NPAD, D), jnp.float32),
    mesh=_MESH,
    scratch_types=(
        [pltpu.VMEM((2, SCH), jnp.int32)] * 8
        + [pltpu.VMEM((SCH, D), jnp.float32)] * 2
        + [pltpu.VMEM_SHARED((NPAD, D), jnp.float32)]
        + [pltpu.SemaphoreType.DMA] * 12
    ),
)
def _scatter_kernel(g_hbm, ec_hbm, zeros_hbm, out_hbm, *scr):
    ibs = scr[0:8]
    rows = scr[8:10]
    acc_sh = scr[10]
    isems = scr[11:19]
    gsems = scr[19:21]
    ssems = scr[21:23]
    c = lax.axis_index("c")
    s = lax.axis_index("s")
    wid = s * NC + c
    r0 = s * ROW_BLK

    @pl.when(s < NS - 1)
    def _():
        pltpu.sync_copy(zeros_hbm.at[pl.ds(0, ROW_BLK)],
                        acc_sh.at[pl.ds(r0, ROW_BLK)])

    @pl.when(s == NS - 1)
    def _():
        pltpu.sync_copy(zeros_hbm, acc_sh.at[pl.ds(r0, SROW_BLK_LAST)])

    plsc.subcore_barrier()

    NCH = SCPW

    def idx_start(j, k):
        pltpu.async_copy(ec_hbm.at[wid, j], ibs[k], isems[k])

    def idx_wait(j, k):
        pltpu.make_async_copy(ec_hbm.at[wid, j], ibs[k], isems[k]).wait()

    def gather_start(k, r):
        pltpu.async_copy(g_hbm.at[ibs[k].at[0]], rows[r], gsems[r])

    def gather_wait(k, r):
        pltpu.make_async_copy(g_hbm.at[ibs[k].at[0]], rows[r], gsems[r]).wait()

    def scat_start(k, r):
        pltpu.async_copy(rows[r], acc_sh.at[ibs[k].at[1]], ssems[r], add=True)

    def scat_wait(k, r):
        pltpu.make_async_copy(rows[r], acc_sh.at[ibs[k].at[1]],
                              ssems[r]).wait()

    for p in range(5):
        idx_start(p, p)
    idx_wait(0, 0)
    gather_start(0, 0)

    def half(j, m):
        k = m % 8
        r = m % 2

        @pl.when((j >= 1) & (j - 1 <= NCH - 1))
        def _():
            scat_wait((m - 1) % 8, (m - 1) % 2)

        @pl.when(j + 5 <= NCH - 1)
        def _():
            idx_start(j + 5, (m + 5) % 8)

        @pl.when(j + 1 <= NCH - 1)
        def _():
            idx_wait(j + 1, (m + 1) % 8)
            gather_start((m + 1) % 8, (m + 1) % 2)

        @pl.when(j <= NCH - 1)
        def _():
            gather_wait(k, r)
            scat_start(k, r)

    def body(i, carry):
        j = 8 * i
        for m in range(8):
            half(j + m, m)
        return carry

    lax.fori_loop(0, (NCH + 8) // 8, body, 0)
    plsc.subcore_barrier()

    @pl.when(s < NS - 1)
    def _():
        pltpu.sync_copy(acc_sh.at[pl.ds(r0, ROW_BLK)],
                        out_hbm.at[c, pl.ds(r0, ROW_BLK)])

    @pl.when(s == NS - 1)
    def _():
        pltpu.sync_copy(acc_sh.at[pl.ds(r0, SROW_BLK_LAST)],
                        out_hbm.at[c, pl.ds(r0, SROW_BLK_LAST)])


_R = 1000


def _dinv_from(deg_ref):
    deg = deg_ref[0, :, 0:1] + deg_ref[1, :, 0:1] + 1.0
    return lax.rsqrt(deg)


def _tc1_body(x_ref, w_ref, deg_ref, g_ref):
    dinv = _dinv_from(deg_ref)
    h = jnp.dot(x_ref[...], w_ref[...], preferred_element_type=jnp.float32)
    g_ref[...] = h * dinv


def _tc2_body(s_ref, g_ref, deg_ref, w_ref, b_ref, out_ref):
    dinv = _dinv_from(deg_ref)
    pre = (s_ref[0] + s_ref[1] + g_ref[...]) * dinv + b_ref[...]
    a = jnp.maximum(pre, 0.0)
    h2 = jnp.dot(a, w_ref[...], preferred_element_type=jnp.float32)
    out_ref[...] = h2 * dinv


def _tc3_body(s_ref, g_ref, deg_ref, b2_ref, wv_ref, bv_ref, wo_ref, bo_ref,
              out_ref):
    dinv = _dinv_from(deg_ref)
    h = (s_ref[0] + s_ref[1] + g_ref[...]) * dinv + b2_ref[...]
    t = jnp.dot(h, wv_ref[...], preferred_element_type=jnp.float32) + bv_ref[...]
    out_ref[...] = (
        jnp.dot(t, wo_ref[...], preferred_element_type=jnp.float32) + bo_ref[...]
    )


_row_spec = pl.BlockSpec((_R, D), lambda i: (i, 0))
_w_spec = pl.BlockSpec((D, D), lambda i: (0, 0))
_b_spec = pl.BlockSpec((1, D), lambda i: (0, 0))
_deg_spec = pl.BlockSpec((NC, _R, DEG_W), lambda i: (0, i, 0))
_s_spec = pl.BlockSpec((NC, _R, D), lambda i: (0, i, 0))
_out_struct = jax.ShapeDtypeStruct((N, D), jnp.float32)
_gpad_struct = jax.ShapeDtypeStruct((NPAD, D), jnp.float32)

_tc1 = pl.pallas_call(
    _tc1_body,
    grid=(N // _R,),
    in_specs=[_row_spec, _w_spec, _deg_spec],
    out_specs=_row_spec,
    out_shape=_gpad_struct,
)

_tc2 = pl.pallas_call(
    _tc2_body,
    grid=(N // _R,),
    in_specs=[_s_spec, _row_spec, _deg_spec, _w_spec, _b_spec],
    out_specs=_row_spec,
    out_shape=_gpad_struct,
)

_tc3 = pl.pallas_call(
    _tc3_body,
    grid=(N // _R,),
    in_specs=[_s_spec, _row_spec, _deg_spec, _b_spec, _w_spec, _b_spec,
              _w_spec, _b_spec],
    out_specs=_row_spec,
    out_shape=_out_struct,
)


def kernel(x, edge_index, W1, b1, W2, b2, Wq, bq, Wk, bk, Wv, bv, Wo, bo):
    pad = jnp.full((NW, EPT - EPW), N, jnp.int32)
    src = jnp.concatenate([edge_index[0].reshape(NW, EPW), pad], axis=1)
    dst = jnp.concatenate([edge_index[1].reshape(NW, EPW), pad], axis=1)
    ec = jnp.concatenate(
        [src.reshape(NW, SCPW, 1, SCH), dst.reshape(NW, SCPW, 1, SCH)],
        axis=2)
    ones_deg = jnp.ones((CHUNK, DEG_W), jnp.float32)
    zeros_deg = jnp.zeros((ROW_BLK_LAST, DEG_W), jnp.float32)
    zeros_s = jnp.zeros((SROW_BLK_LAST, D), jnp.float32)

    degt = _deg_kernel(edge_index[1].reshape(NW, CHUNKS_PER_W, CHUNK),
                       ones_deg, zeros_deg)
    g1 = _tc1(x, W1, degt)
    s1 = _scatter_kernel(g1, ec, zeros_s)
    g2 = _tc2(s1, g1, degt, W2, b1.reshape(1, D))
    s2 = _scatter_kernel(g2, ec, zeros_s)
    out = _tc3(s2, g2, degt, b2.reshape(1, D), Wv, bv.reshape(1, D),
               Wo, bo.reshape(1, D))
    return out.reshape(N, 1, D)


# speedup vs baseline: 1.7260x; 1.7260x over previous
import functools

import jax
import jax.numpy as jnp
from jax import lax
from jax.experimental import pallas as pl
from jax.experimental.pallas import tpu as pltpu
from jax.experimental.pallas import tpu_sc as plsc

N = 10000
E = 320000
D = 128
NC = 2
NS = 16
NW = NC * NS
CHUNK = 80
CHUNKS_PER_W = E // (NW * CHUNK)
ROW_BLK = 624
ROW_BLK_LAST = N - (NS - 1) * ROW_BLK
SCH = 128
EPW = E // NW
EPT = SCH * (-(-EPW // SCH))
SCPW = EPT // SCH
NPAD = N + 8
SROW_BLK_LAST = NPAD - (NS - 1) * ROW_BLK
DEG_W = 16

_MESH = plsc.VectorSubcoreMesh(
    core_axis_name="c", subcore_axis_name="s", num_cores=NC, num_subcores=NS
)


@functools.partial(
    pl.kernel,
    out_type=jax.ShapeDtypeStruct((NC, N, DEG_W), jnp.float32),
    mesh=_MESH,
    compiler_params=pltpu.CompilerParams(use_tc_tiling_on_sc=False),
    scratch_types=[
        pltpu.VMEM((CHUNKS_PER_W, CHUNK), jnp.int32),
        pltpu.VMEM((CHUNK, DEG_W), jnp.float32),
        pltpu.VMEM_SHARED((N, DEG_W), jnp.float32),
        pltpu.SemaphoreType.DMA,
    ],
)
def _deg_kernel(dst_hbm, ones_hbm, zeros_hbm, out_hbm, idx_v, ones_v, acc_sh,
                sem):
    c = lax.axis_index("c")
    s = lax.axis_index("s")
    wid = s * NC + c
    r0 = s * ROW_BLK

    @pl.when(s < NS - 1)
    def _():
        pltpu.sync_copy(zeros_hbm.at[pl.ds(0, ROW_BLK)],
                        acc_sh.at[pl.ds(r0, ROW_BLK)])

    @pl.when(s == NS - 1)
    def _():
        pltpu.sync_copy(zeros_hbm, acc_sh.at[pl.ds(r0, ROW_BLK_LAST)])

    pltpu.sync_copy(ones_hbm, ones_v)
    pltpu.sync_copy(dst_hbm.at[wid], idx_v)
    plsc.subcore_barrier()

    WINDOW = 8

    def body(i, carry):
        @pl.when(i >= WINDOW)
        def _():
            pltpu.make_async_copy(ones_v, acc_sh.at[idx_v.at[0]], sem).wait()

        pltpu.async_copy(ones_v, acc_sh.at[idx_v.at[i]], sem, add=True)
        return carry

    lax.fori_loop(0, CHUNKS_PER_W, body, 0)

    def drain(i, carry):
        pltpu.make_async_copy(ones_v, acc_sh.at[idx_v.at[0]], sem).wait()
        return carry

    lax.fori_loop(0, WINDOW, drain, 0)
    plsc.subcore_barrier()

    @pl.when(s < NS - 1)
    def _():
        pltpu.sync_copy(acc_sh.at[pl.ds(r0, ROW_BLK)],
                        out_hbm.at[c, pl.ds(r0, ROW_BLK)])

    @pl.when(s == NS - 1)
    def _():
        pltpu.sync_copy(acc_sh.at[pl.ds(r0, ROW_BLK_LAST)],
                        out_hbm.at[c, pl.ds(r0, ROW_BLK_LAST)])


@functools.partial(
    pl.kernel,
    out_type=jax.ShapeDtypeStruct((NC, N, D), jnp.float32),
    mesh=_MESH,
    scratch_types=(
        [pltpu.VMEM((2, CHUNK), jnp.int32)] * 8
        + [pltpu.VMEM((CHUNK, D), jnp.float32)] * 4
        + [pltpu.VMEM_SHARED((N, D), jnp.float32)]
        + [pltpu.SemaphoreType.DMA] * 16
    ),
)
def _scatter_kernel(g_hbm, ec_hbm, zeros_hbm, out_hbm, *scr):
    ibs = scr[0:8]
    rows = scr[8:12]
    acc_sh = scr[12]
    isems = scr[13:21]
    gsems = scr[21:25]
    ssems = scr[25:29]
    c = lax.axis_index("c")
    s = lax.axis_index("s")
    wid = s * NC + c
    r0 = s * ROW_BLK

    @pl.when(s < NS - 1)
    def _():
        pltpu.sync_copy(zeros_hbm.at[pl.ds(0, ROW_BLK)],
                        acc_sh.at[pl.ds(r0, ROW_BLK)])

    @pl.when(s == NS - 1)
    def _():
        pltpu.sync_copy(zeros_hbm, acc_sh.at[pl.ds(r0, ROW_BLK_LAST)])

    plsc.subcore_barrier()

    NCH = CHUNKS_PER_W

    def idx_start(j, k):
        pltpu.async_copy(ec_hbm.at[wid, j], ibs[k], isems[k])

    def idx_wait(j, k):
        pltpu.make_async_copy(ec_hbm.at[wid, j], ibs[k], isems[k]).wait()

    def gather_start(k, r):
        pltpu.async_copy(g_hbm.at[ibs[k].at[0]], rows[r], gsems[r])

    def gather_wait(k, r):
        pltpu.make_async_copy(g_hbm.at[ibs[k].at[0]], rows[r], gsems[r]).wait()

    def scat_start(k, r):
        pltpu.async_copy(rows[r], acc_sh.at[ibs[k].at[1]], ssems[r], add=True)

    def scat_wait(k, r):
        pltpu.make_async_copy(rows[r], acc_sh.at[ibs[k].at[1]],
                              ssems[r]).wait()

    for p in range(5):
        idx_start(p, p)
    idx_wait(0, 0)
    gather_start(0, 0)

    def half(j, m):
        k = m % 8
        r = m % 4

        @pl.when((j >= 3) & (j - 3 <= NCH - 1))
        def _():
            scat_wait((m - 3) % 8, (m - 3) % 4)

        @pl.when(j + 5 <= NCH - 1)
        def _():
            idx_start(j + 5, (m + 5) % 8)

        @pl.when(j + 1 <= NCH - 1)
        def _():
            idx_wait(j + 1, (m + 1) % 8)
            gather_start((m + 1) % 8, (m + 1) % 4)

        @pl.when(j <= NCH - 1)
        def _():
            gather_wait(k, r)
            scat_start(k, r)

    def body(i, carry):
        j = 8 * i
        for m in range(8):
            half(j + m, m)
        return carry

    lax.fori_loop(0, (NCH + 7) // 8, body, 0)
    plsc.subcore_barrier()

    @pl.when(s < NS - 1)
    def _():
        pltpu.sync_copy(acc_sh.at[pl.ds(r0, ROW_BLK)],
                        out_hbm.at[c, pl.ds(r0, ROW_BLK)])

    @pl.when(s == NS - 1)
    def _():
        pltpu.sync_copy(acc_sh.at[pl.ds(r0, ROW_BLK_LAST)],
                        out_hbm.at[c, pl.ds(r0, ROW_BLK_LAST)])


_R = 1000


def _dinv_from(deg_ref):
    deg = deg_ref[0, :, 0:1] + deg_ref[1, :, 0:1] + 1.0
    return lax.rsqrt(deg)


def _tc1_body(x_ref, w_ref, deg_ref, g_ref):
    dinv = _dinv_from(deg_ref)
    h = jnp.dot(x_ref[...], w_ref[...], preferred_element_type=jnp.float32)
    g_ref[...] = h * dinv


def _tc2_body(s_ref, g_ref, deg_ref, w_ref, b_ref, out_ref):
    dinv = _dinv_from(deg_ref)
    pre = (s_ref[0] + s_ref[1] + g_ref[...]) * dinv + b_ref[...]
    a = jnp.maximum(pre, 0.0)
    h2 = jnp.dot(a, w_ref[...], preferred_element_type=jnp.float32)
    out_ref[...] = h2 * dinv


def _tc3_body(s_ref, g_ref, deg_ref, b2_ref, wv_ref, bv_ref, wo_ref, bo_ref,
              out_ref):
    dinv = _dinv_from(deg_ref)
    h = (s_ref[0] + s_ref[1] + g_ref[...]) * dinv + b2_ref[...]
    t = jnp.dot(h, wv_ref[...], preferred_element_type=jnp.float32) + bv_ref[...]
    out_ref[...] = (
        jnp.dot(t, wo_ref[...], preferred_element_type=jnp.float32) + bo_ref[...]
    )


_row_spec = pl.BlockSpec((_R, D), lambda i: (i, 0))
_w_spec = pl.BlockSpec((D, D), lambda i: (0, 0))
_b_spec = pl.BlockSpec((1, D), lambda i: (0, 0))
_deg_spec = pl.BlockSpec((NC, _R, DEG_W), lambda i: (0, i, 0))
_s_spec = pl.BlockSpec((NC, _R, D), lambda i: (0, i, 0))
_out_struct = jax.ShapeDtypeStruct((N, D), jnp.float32)
_gpad_struct = _out_struct

_tc1 = pl.pallas_call(
    _tc1_body,
    grid=(N // _R,),
    in_specs=[_row_spec, _w_spec, _deg_spec],
    out_specs=_row_spec,
    out_shape=_gpad_struct,
)

_tc2 = pl.pallas_call(
    _tc2_body,
    grid=(N // _R,),
    in_specs=[_s_spec, _row_spec, _deg_spec, _w_spec, _b_spec],
    out_specs=_row_spec,
    out_shape=_gpad_struct,
)

_tc3 = pl.pallas_call(
    _tc3_body,
    grid=(N // _R,),
    in_specs=[_s_spec, _row_spec, _deg_spec, _b_spec, _w_spec, _b_spec,
              _w_spec, _b_spec],
    out_specs=_row_spec,
    out_shape=_out_struct,
)


def kernel(x, edge_index, W1, b1, W2, b2, Wq, bq, Wk, bk, Wv, bv, Wo, bo):
    ec = jnp.concatenate(
        [edge_index[0].reshape(NW, CHUNKS_PER_W, 1, CHUNK),
         edge_index[1].reshape(NW, CHUNKS_PER_W, 1, CHUNK)],
        axis=2)
    ones_deg = jnp.ones((CHUNK, DEG_W), jnp.float32)
    zeros_deg = jnp.zeros((ROW_BLK_LAST, DEG_W), jnp.float32)
    zeros_s = jnp.zeros((ROW_BLK_LAST, D), jnp.float32)

    degt = _deg_kernel(edge_index[1].reshape(NW, CHUNKS_PER_W, CHUNK),
                       ones_deg, zeros_deg)
    g1 = _tc1(x, W1, degt)
    s1 = _scatter_kernel(g1, ec, zeros_s)
    g2 = _tc2(s1, g1, degt, W2, b1.reshape(1, D))
    s2 = _scatter_kernel(g2, ec, zeros_s)
    out = _tc3(s2, g2, degt, b2.reshape(1, D), Wv, bv.reshape(1, D),
               Wo, bo.reshape(1, D))
    return out.reshape(N, 1, D)


# speedup vs baseline: 1.7696x; 1.0253x over previous
import functools

import jax
import jax.numpy as jnp
from jax import lax
from jax.experimental import pallas as pl
from jax.experimental.pallas import tpu as pltpu
from jax.experimental.pallas import tpu_sc as plsc

N = 10000
E = 320000
D = 128
NC = 2
NS = 16
NW = NC * NS
CHUNK = 80
CHUNKS_PER_W = E // (NW * CHUNK)
ROW_BLK = 624
ROW_BLK_LAST = N - (NS - 1) * ROW_BLK
SCH = 128
EPW = E // NW
EPT = SCH * (-(-EPW // SCH))
SCPW = EPT // SCH
NPAD = N + 8
SROW_BLK_LAST = NPAD - (NS - 1) * ROW_BLK
DEG_W = 16

_MESH = plsc.VectorSubcoreMesh(
    core_axis_name="c", subcore_axis_name="s", num_cores=NC, num_subcores=NS
)


@functools.partial(
    pl.kernel,
    out_type=jax.ShapeDtypeStruct((NC, N, DEG_W), jnp.float32),
    mesh=_MESH,
    compiler_params=pltpu.CompilerParams(use_tc_tiling_on_sc=False),
    scratch_types=[
        pltpu.VMEM((CHUNKS_PER_W, CHUNK), jnp.int32),
        pltpu.VMEM((CHUNK, DEG_W), jnp.float32),
        pltpu.VMEM_SHARED((N, DEG_W), jnp.float32),
        pltpu.SemaphoreType.DMA,
    ],
)
def _deg_kernel(dst_hbm, ones_hbm, zeros_hbm, out_hbm, idx_v, ones_v, acc_sh,
                sem):
    c = lax.axis_index("c")
    s = lax.axis_index("s")
    wid = s * NC + c
    r0 = s * ROW_BLK

    @pl.when(s < NS - 1)
    def _():
        pltpu.sync_copy(zeros_hbm.at[pl.ds(0, ROW_BLK)],
                        acc_sh.at[pl.ds(r0, ROW_BLK)])

    @pl.when(s == NS - 1)
    def _():
        pltpu.sync_copy(zeros_hbm, acc_sh.at[pl.ds(r0, ROW_BLK_LAST)])

    pltpu.sync_copy(ones_hbm, ones_v)
    pltpu.sync_copy(dst_hbm.at[wid], idx_v)
    plsc.subcore_barrier()

    WINDOW = 8

    def body(i, carry):
        @pl.when(i >= WINDOW)
        def _():
            pltpu.make_async_copy(ones_v, acc_sh.at[idx_v.at[0]], sem).wait()

        pltpu.async_copy(ones_v, acc_sh.at[idx_v.at[i]], sem, add=True)
        return carry

    lax.fori_loop(0, CHUNKS_PER_W, body, 0)

    def drain(i, carry):
        pltpu.make_async_copy(ones_v, acc_sh.at[idx_v.at[0]], sem).wait()
        return carry

    lax.fori_loop(0, WINDOW, drain, 0)
    plsc.subcore_barrier()

    @pl.when(s < NS - 1)
    def _():
        pltpu.sync_copy(acc_sh.at[pl.ds(r0, ROW_BLK)],
                        out_hbm.at[c, pl.ds(r0, ROW_BLK)])

    @pl.when(s == NS - 1)
    def _():
        pltpu.sync_copy(acc_sh.at[pl.ds(r0, ROW_BLK_LAST)],
                        out_hbm.at[c, pl.ds(r0, ROW_BLK_LAST)])


@functools.partial(
    pl.kernel,
    out_type=jax.ShapeDtypeStruct((NC, N, D), jnp.float32),
    mesh=_MESH,
    scratch_types=(
        [pltpu.VMEM((2, CHUNK), jnp.int32)] * 8
        + [pltpu.VMEM((CHUNK, D), jnp.float32)] * 4
        + [pltpu.VMEM_SHARED((N, D), jnp.float32)]
        + [pltpu.SemaphoreType.DMA] * 16
    ),
)
def _scatter_kernel(g_hbm, ec_hbm, zeros_hbm, out_hbm, *scr):
    ibs = scr[0:8]
    rows = scr[8:12]
    acc_sh = scr[12]
    isems = scr[13:21]
    gsems = scr[21:25]
    ssems = scr[25:29]
    c = lax.axis_index("c")
    s = lax.axis_index("s")
    wid = s * NC + c
    r0 = s * ROW_BLK

    @pl.when(s < NS - 1)
    def _():
        pltpu.sync_copy(zeros_hbm.at[pl.ds(0, ROW_BLK)],
                        acc_sh.at[pl.ds(r0, ROW_BLK)])

    @pl.when(s == NS - 1)
    def _():
        pltpu.sync_copy(zeros_hbm, acc_sh.at[pl.ds(r0, ROW_BLK_LAST)])

    plsc.subcore_barrier()

    NCH = CHUNKS_PER_W

    def idx_start(j, k):
        pltpu.async_copy(ec_hbm.at[wid, j], ibs[k], isems[k])

    def idx_wait(j, k):
        pltpu.make_async_copy(ec_hbm.at[wid, j], ibs[k], isems[k]).wait()

    def gather_start(k, r):
        pltpu.async_copy(g_hbm.at[ibs[k].at[0]], rows[r], gsems[r])

    def gather_wait(k, r):
        pltpu.make_async_copy(g_hbm.at[ibs[k].at[0]], rows[r], gsems[r]).wait()

    def scat_start(k, r):
        pltpu.async_copy(rows[r], acc_sh.at[ibs[k].at[1]], ssems[r], add=True)

    def scat_wait(k, r):
        pltpu.make_async_copy(rows[r], acc_sh.at[ibs[k].at[1]],
                              ssems[r]).wait()

    for p in range(5):
        idx_start(p, p)
    idx_wait(0, 0)
    gather_start(0, 0)

    def half(j, m):
        k = m % 8
        r = m % 4

        @pl.when((j >= 3) & (j - 3 <= NCH - 1))
        def _():
            scat_wait((m - 3) % 8, (m - 3) % 4)

        @pl.when(j + 5 <= NCH - 1)
        def _():
            idx_start(j + 5, (m + 5) % 8)

        @pl.when(j + 1 <= NCH - 1)
        def _():
            idx_wait(j + 1, (m + 1) % 8)
            gather_start((m + 1) % 8, (m + 1) % 4)

        @pl.when(j <= NCH - 1)
        def _():
            gather_wait(k, r)
            scat_start(k, r)

    def body(i, carry):
        j = 8 * i
        for m in range(8):
            half(j + m, m)
        return carry

    lax.fori_loop(0, (NCH + 7) // 8, body, 0)
    plsc.subcore_barrier()

    @pl.when(s < NS - 1)
    def _():
        pltpu.sync_copy(acc_sh.at[pl.ds(r0, ROW_BLK)],
                        out_hbm.at[c, pl.ds(r0, ROW_BLK)])

    @pl.when(s == NS - 1)
    def _():
        pltpu.sync_copy(acc_sh.at[pl.ds(r0, ROW_BLK_LAST)],
                        out_hbm.at[c, pl.ds(r0, ROW_BLK_LAST)])


_R = 2000


def _dinv_from(deg_ref):
    deg = deg_ref[0, :, 0:1] + deg_ref[1, :, 0:1] + 1.0
    return lax.rsqrt(deg)


def _tc1_body(x_ref, w_ref, deg_ref, g_ref):
    dinv = _dinv_from(deg_ref)
    h = jnp.dot(x_ref[...], w_ref[...], preferred_element_type=jnp.float32)
    g_ref[...] = h * dinv


def _tc2_body(s_ref, g_ref, deg_ref, w_ref, b_ref, out_ref):
    dinv = _dinv_from(deg_ref)
    pre = (s_ref[0] + s_ref[1] + g_ref[...]) * dinv + b_ref[...]
    a = jnp.maximum(pre, 0.0)
    h2 = jnp.dot(a, w_ref[...], preferred_element_type=jnp.float32)
    out_ref[...] = h2 * dinv


def _tc3_body(s_ref, g_ref, deg_ref, b2_ref, wv_ref, bv_ref, wo_ref, bo_ref,
              out_ref):
    dinv = _dinv_from(deg_ref)
    h = (s_ref[0] + s_ref[1] + g_ref[...]) * dinv + b2_ref[...]
    t = jnp.dot(h, wv_ref[...], preferred_element_type=jnp.float32) + bv_ref[...]
    out_ref[...] = (
        jnp.dot(t, wo_ref[...], preferred_element_type=jnp.float32) + bo_ref[...]
    )


_row_spec = pl.BlockSpec((_R, D), lambda i: (i, 0))
_w_spec = pl.BlockSpec((D, D), lambda i: (0, 0))
_b_spec = pl.BlockSpec((1, D), lambda i: (0, 0))
_deg_spec = pl.BlockSpec((NC, _R, DEG_W), lambda i: (0, i, 0))
_s_spec = pl.BlockSpec((NC, _R, D), lambda i: (0, i, 0))
_out_struct = jax.ShapeDtypeStruct((N, D), jnp.float32)
_gpad_struct = _out_struct

_tc1 = pl.pallas_call(
    _tc1_body,
    grid=(N // _R,),
    in_specs=[_row_spec, _w_spec, _deg_spec],
    out_specs=_row_spec,
    out_shape=_gpad_struct,
)

_tc2 = pl.pallas_call(
    _tc2_body,
    grid=(N // _R,),
    in_specs=[_s_spec, _row_spec, _deg_spec, _w_spec, _b_spec],
    out_specs=_row_spec,
    out_shape=_gpad_struct,
)

_tc3 = pl.pallas_call(
    _tc3_body,
    grid=(N // _R,),
    in_specs=[_s_spec, _row_spec, _deg_spec, _b_spec, _w_spec, _b_spec,
              _w_spec, _b_spec],
    out_specs=_row_spec,
    out_shape=_out_struct,
)


def kernel(x, edge_index, W1, b1, W2, b2, Wq, bq, Wk, bk, Wv, bv, Wo, bo):
    ec = jnp.concatenate(
        [edge_index[0].reshape(NW, CHUNKS_PER_W, 1, CHUNK),
         edge_index[1].reshape(NW, CHUNKS_PER_W, 1, CHUNK)],
        axis=2)
    ones_deg = jnp.ones((CHUNK, DEG_W), jnp.float32)
    zeros_deg = jnp.zeros((ROW_BLK_LAST, DEG_W), jnp.float32)
    zeros_s = jnp.zeros((ROW_BLK_LAST, D), jnp.float32)

    degt = _deg_kernel(edge_index[1].reshape(NW, CHUNKS_PER_W, CHUNK),
                       ones_deg, zeros_deg)
    g1 = _tc1(x, W1, degt)
    s1 = _scatter_kernel(g1, ec, zeros_s)
    g2 = _tc2(s1, g1, degt, W2, b1.reshape(1, D))
    s2 = _scatter_kernel(g2, ec, zeros_s)
    out = _tc3(s2, g2, degt, b2.reshape(1, D), Wv, bv.reshape(1, D),
               Wo, bo.reshape(1, D))
    return out.reshape(N, 1, D)
